# baseline (speedup 1.0000x reference)
import functools

import jax
import jax.numpy as jnp
from jax import lax
from jax.experimental import pallas as pl
from jax.experimental.pallas import tpu as pltpu
from jax.experimental.pallas import tpu_sc as plsc

NUM_CLASSES = 1000000
EMBEDDING_DIM = 64
BATCH = 16384

_info = plsc.get_sparse_core_info()
_NUM_WORKERS = _info.num_cores * _info.num_subcores
_B_PER_W = BATCH // _NUM_WORKERS


@functools.partial(
    jax.jit,
    static_argnames=(),
)
def _gather(labels, table):
    mesh = plsc.VectorSubcoreMesh(core_axis_name="c", subcore_axis_name="s")

    @functools.partial(
        pl.kernel,
        mesh=mesh,
        out_type=jax.ShapeDtypeStruct((BATCH, EMBEDDING_DIM), jnp.float32),
        scratch_types=[
            pltpu.VMEM((_B_PER_W,), jnp.int32),
            pltpu.VMEM((_B_PER_W, EMBEDDING_DIM), jnp.float32),
            pltpu.SemaphoreType.DMA,
        ],
        compiler_params=pltpu.CompilerParams(use_tc_tiling_on_sc=False),
    )
    def k(labels_hbm, table_hbm, out_hbm, idx_v, rows_v, sem):
        wid = lax.axis_index("s") * _info.num_cores + lax.axis_index("c")
        base = wid * _B_PER_W
        pltpu.sync_copy(labels_hbm.at[pl.ds(base, _B_PER_W)], idx_v)
        pltpu.async_copy(table_hbm.at[idx_v], rows_v, sem).wait()
        pltpu.sync_copy(rows_v, out_hbm.at[pl.ds(base, _B_PER_W)])

    return k(labels, table)


def kernel(labels, table):
    return _gather(labels.astype(jnp.int32), table)


# speedup vs baseline: 1.0012x; 1.0012x over previous
import functools

import jax
import jax.numpy as jnp
from jax import lax
from jax.experimental import pallas as pl
from jax.experimental.pallas import tpu as pltpu
from jax.experimental.pallas import tpu_sc as plsc

NUM_CLASSES = 1000000
EMBEDDING_DIM = 64
BATCH = 16384

_info = plsc.get_sparse_core_info()
_NUM_WORKERS = _info.num_cores * _info.num_subcores
_B_PER_W = BATCH // _NUM_WORKERS


@functools.partial(
    jax.jit,
    static_argnames=(),
)
def _gather(labels, table):
    mesh = plsc.VectorSubcoreMesh(core_axis_name="c", subcore_axis_name="s")

    n_chunks = 4
    chunk = _B_PER_W // n_chunks

    @functools.partial(
        pl.kernel,
        mesh=mesh,
        out_type=jax.ShapeDtypeStruct((BATCH, EMBEDDING_DIM), jnp.float32),
        scratch_types=[
            pltpu.VMEM((_B_PER_W,), jnp.int32),
            pltpu.VMEM((_B_PER_W, EMBEDDING_DIM), jnp.float32),
            pltpu.SemaphoreType.DMA((n_chunks,)),
            pltpu.SemaphoreType.DMA,
        ],
        compiler_params=pltpu.CompilerParams(use_tc_tiling_on_sc=False),
    )
    def k(labels_hbm, table_hbm, out_hbm, idx_v, rows_v, gsems, ssem):
        wid = lax.axis_index("s") * _info.num_cores + lax.axis_index("c")
        base = wid * _B_PER_W
        pltpu.sync_copy(labels_hbm.at[pl.ds(base, _B_PER_W)], idx_v)
        gathers = []
        for c in range(n_chunks):
            gathers.append(
                pltpu.async_copy(
                    table_hbm.at[idx_v.at[pl.ds(c * chunk, chunk)]],
                    rows_v.at[pl.ds(c * chunk, chunk)],
                    gsems.at[c],
                )
            )
        scatters = []
        for c in range(n_chunks):
            gathers[c].wait()
            scatters.append(
                pltpu.async_copy(
                    rows_v.at[pl.ds(c * chunk, chunk)],
                    out_hbm.at[pl.ds(base + c * chunk, chunk)],
                    ssem,
                )
            )
        for c in range(n_chunks):
            scatters[c].wait()

    return k(labels, table)


def kernel(labels, table):
    return _gather(labels.astype(jnp.int32), table)


# speedup vs baseline: 1.7221x; 1.7200x over previous
import functools

import jax
import jax.numpy as jnp
from jax import lax
from jax.experimental import pallas as pl
from jax.experimental.pallas import tpu as pltpu
from jax.experimental.pallas import tpu_sc as plsc

NUM_CLASSES = 1000000
EMBEDDING_DIM = 64
BATCH = 16384

_info = plsc.get_sparse_core_info()
_NUM_WORKERS = _info.num_cores * _info.num_subcores
_B_PER_W = BATCH // _NUM_WORKERS

_GROUP = 64
_N_GROUPS = _B_PER_W // _GROUP


def _gather(labels, table):
    mesh = plsc.VectorSubcoreMesh(core_axis_name="c", subcore_axis_name="s")

    @functools.partial(
        pl.kernel,
        mesh=mesh,
        out_type=jax.ShapeDtypeStruct((BATCH, EMBEDDING_DIM), jnp.float32),
        scratch_types=[
            pltpu.VMEM((_B_PER_W,), jnp.int32),
            pltpu.VMEM((_B_PER_W, EMBEDDING_DIM), jnp.float32),
            pltpu.SemaphoreType.DMA((_N_GROUPS,)),
            pltpu.SemaphoreType.DMA,
        ],
    )
    def k(labels_hbm, table_hbm, out_hbm, idx_v, rows_v, gsems, ssem):
        wid = lax.axis_index("s") * _info.num_cores + lax.axis_index("c")
        base = wid * _B_PER_W
        pltpu.sync_copy(labels_hbm.at[pl.ds(base, _B_PER_W)], idx_v)

        def fire(g):
            for v16 in range(_GROUP // 16):
                jb = g * _GROUP + v16 * 16
                vec = idx_v[pl.ds(jb, 16)]
                for u in range(16):
                    r = vec[u]
                    pltpu.async_copy(
                        table_hbm.at[pl.ds(r, 1)],
                        rows_v.at[pl.ds(jb + u, 1)],
                        gsems.at[g],
                    )

        def drain_rows(g):
            pltpu.make_async_copy(
                table_hbm.at[pl.ds(0, _GROUP)],
                rows_v.at[pl.ds(g * _GROUP, _GROUP)],
                gsems.at[g],
            ).wait()

        fire(0)

        def body(g, carry):
            @pl.when(g + 1 < _N_GROUPS)
            def _():
                fire(g + 1)

            drain_rows(g)
            pltpu.async_copy(
                rows_v.at[pl.ds(g * _GROUP, _GROUP)],
                out_hbm.at[pl.ds(base + g * _GROUP, _GROUP)],
                ssem,
            )
            return carry

        lax.fori_loop(0, _N_GROUPS, body, 0)

        pltpu.make_async_copy(
            rows_v, out_hbm.at[pl.ds(base, _B_PER_W)], ssem
        ).wait()

    return k(labels, table)


def kernel(labels, table):
    return _gather(labels.astype(jnp.int32), table)
